# baseline (speedup 1.0000x reference)
import jax
import jax.numpy as jnp
from jax import lax
from jax.experimental import pallas as pl
from jax.experimental.pallas import tpu as pltpu
from jax.experimental.pallas import tpu_sc as plsc

F = 26
V = 100000
D = 16
B = 16384

N = B * F
NC, NS, L = 2, 16, 16
NW = NC * NS
PW = N // NW
CH = 128
NG = PW // CH
GPS = 8
NSTG = NG // GPS
SROWS = GPS * CH


def _body(xf_hbm, tab_hbm, out_hbm, idx_v, rows_v, gsem, osem):
    wid = lax.axis_index("s") * NC + lax.axis_index("c")
    row0 = wid * NG
    base = wid * PW

    pltpu.sync_copy(xf_hbm.at[pl.ds(row0, NG)], idx_v)

    lane = lax.iota(jnp.int32, L)

    def add_off(j, carry):
        for k in range(CH // L):
            pos = base + j * CH + k * L + lane
            off = lax.rem(pos, F) * V
            idx_v[j, pl.ds(k * L, L)] = idx_v[j, pl.ds(k * L, L)] + off
        return carry

    lax.fori_loop(0, NG, add_off, 0)

    def fire(s, buf):
        for g in range(GPS):
            pltpu.async_copy(
                tab_hbm.at[idx_v.at[s * GPS + g]],
                rows_v.at[buf, pl.ds(g * CH, CH)],
                gsem,
            )

    def drain(buf):
        for g in range(GPS):
            pltpu.make_async_copy(
                tab_hbm.at[idx_v.at[g]],
                rows_v.at[buf, pl.ds(g * CH, CH)],
                gsem,
            ).wait()

    fire(0, 0)
    for s in range(NSTG):
        buf = s % 2
        drain(buf)
        if s + 1 < NSTG:
            fire(s + 1, 1 - buf)
        pltpu.async_copy(
            rows_v.at[buf],
            out_hbm.at[pl.ds(base + s * SROWS, SROWS)],
            osem,
        ).wait()


def _sc_gather(xf, tab):
    mesh = plsc.VectorSubcoreMesh(core_axis_name="c", subcore_axis_name="s")
    return pl.kernel(
        _body,
        out_type=jax.ShapeDtypeStruct((N, D), jnp.float32),
        mesh=mesh,
        scratch_types=[
            pltpu.VMEM((NG, CH), jnp.int32),
            pltpu.VMEM((2, SROWS, D), jnp.float32),
            pltpu.SemaphoreType.DMA,
            pltpu.SemaphoreType.DMA,
        ],
        compiler_params=pltpu.CompilerParams(use_tc_tiling_on_sc=False),
    )(xf, tab)


def kernel(X, tables):
    xf = X.reshape(N // CH, CH)
    tab = tables.reshape(F * V, D)
    out = _sc_gather(xf, tab)
    return out.reshape(B, F, D)


# speedup vs baseline: 2.6498x; 2.6498x over previous
import jax
import jax.numpy as jnp
from jax import lax
from jax.experimental import pallas as pl
from jax.experimental.pallas import tpu as pltpu
from jax.experimental.pallas import tpu_sc as plsc

F = 26
V = 100000
D = 16
B = 16384

NC, NS, L = 2, 16, 16
NW = NC * NS
NTF = 781
VREM = V - NTF * 128
TPF = NTF + 1
SROWS = F * TPF * D
NB1 = F * NTF
NA1 = 636
BPW = B // NW
NBLK = F * (BPW // 128)
XW = BPW * F
ROWS2 = F * 2 * (B // 128) * 8


def _detile_body(tabT_hbm, tail_hbm, scr_hbm, vin0, vin1, vtail,
                 si0, si1, so0, so1):
    c = lax.axis_index("c")
    t = lax.axis_index("s")
    w = t * NC + c
    lax.iota(jnp.int32, L)

    def act(i):
        return (w + i * NW) < NB1

    def f_vt(i):
        bid = w + i * NW
        f = bid // NTF
        return f, bid - f * NTF

    def start_in(i, vin, sem):
        @pl.when(act(i))
        def _():
            f, vt = f_vt(i)
            pltpu.async_copy(
                tabT_hbm.at[f, :, pl.ds(vt * 128, 128)], vin, sem)

    def wait_in(i, vin, sem):
        @pl.when(act(i))
        def _():
            pltpu.make_async_copy(
                tabT_hbm.at[0, :, pl.ds(0, 128)], vin, sem).wait()

    def start_out(i, vin, sem):
        @pl.when(act(i))
        def _():
            f, vt = f_vt(i)
            row0 = (f * TPF + vt) * D
            pltpu.async_copy(vin, scr_hbm.at[pl.ds(row0, D), :], sem)

    def wait_out(pred, vin, sem):
        @pl.when(pred)
        def _():
            pltpu.make_async_copy(
                vin, scr_hbm.at[pl.ds(0, D), :], sem).wait()

    start_in(0, vin0, si0)

    def pair(ip, carry):
        for par in range(2):
            i = ip * 2 + par
            vin, sin, sout = (vin0, si0, so0) if par == 0 else (vin1, si1, so1)
            nvin, nsin, nsout = (vin1, si1, so1) if par == 0 else (vin0, si0, so0)
            wait_out((i >= 1) & act(i - 1), nvin, nsout)
            start_in(i + 1, nvin, nsin)
            wait_in(i, vin, sin)
            start_out(i, vin, sout)
        return carry

    lax.fori_loop(0, NA1 // 2, pair, 0)
    wait_out(act(NA1 - 1), vin1, so1)

    @pl.when(w < F)
    def _():
        pltpu.sync_copy(tail_hbm.at[pl.ds(w * 8, 8), :], vtail)
        row0 = (w * TPF + NTF) * D
        pltpu.sync_copy(vtail, scr_hbm.at[pl.ds(row0, 8), :])


def _detile(tabT, tailp):
    mesh = plsc.VectorSubcoreMesh(core_axis_name="c", subcore_axis_name="s")
    return pl.kernel(
        _detile_body,
        out_type=jax.ShapeDtypeStruct((SROWS, 128), jnp.float32),
        mesh=mesh,
        scratch_types=[
            pltpu.VMEM((D, 128), jnp.float32),
            pltpu.VMEM((D, 128), jnp.float32),
            pltpu.VMEM((8, 128), jnp.float32),
            pltpu.SemaphoreType.DMA,
            pltpu.SemaphoreType.DMA,
            pltpu.SemaphoreType.DMA,
            pltpu.SemaphoreType.DMA,
        ],
        compiler_params=pltpu.CompilerParams(use_tc_tiling_on_sc=True),
    )(tabT, tailp)


def _gather_body(xf_hbm, scr_hbm, out_hbm,
                 xblk, idx0, idx1, wout0, wout1,
                 sg0, sg1, sw0, sw1):
    c = lax.axis_index("c")
    t = lax.axis_index("s")
    w = t * NC + c
    lane = lax.iota(jnp.int32, L)

    pltpu.sync_copy(xf_hbm.at[pl.ds(w * XW, XW)], xblk)

    def split(i):
        f = i // 4
        btl = i - f * 4
        return f, w * 4 + btl, btl * 128

    def build_idx(i, idxm):
        f, _, b0 = split(i)
        for k in range(8):
            pos = (b0 + k * 16 + lane) * F + f
            v = plsc.load_gather(xblk, [pos])
            vt = lax.shift_right_logical(v, 7)
            base = (f * TPF + vt) * (D * 128) + (v & 127)
            tmask = v >= (NTF * 128)
            tbase = (f * TPF + NTF) * (D * 128) + (v - NTF * 128) * D
            for e in range(D):
                idxm[e, pl.ds(k * 16, L)] = jnp.where(
                    tmask, tbase + e, base + e * 128)

    def start_gathers(idxm, wm, sem):
        for e in range(D):
            pltpu.async_copy(scr_hbm.at[idxm.at[e]], wm.at[e], sem)

    def wait_gathers(idxm, wm, sem):
        for e in range(D):
            pltpu.make_async_copy(scr_hbm.at[idxm.at[e]], wm.at[e], sem).wait()

    def start_out(i, wm, sem):
        f, bt, _ = split(i)
        pltpu.async_copy(
            wm.at[pl.ds(0, 8), :],
            out_hbm.at[pl.ds(((f * 2 + 0) * 128 + bt) * 8, 8), :], sem)
        pltpu.async_copy(
            wm.at[pl.ds(8, 8), :],
            out_hbm.at[pl.ds(((f * 2 + 1) * 128 + bt) * 8, 8), :], sem)

    def wait_out(wm, sem):
        for _ in range(2):
            pltpu.make_async_copy(
                wm.at[pl.ds(0, 8), :],
                out_hbm.at[pl.ds(0, 8), :], sem).wait()

    build_idx(0, idx0)
    start_gathers(idx0, wout0, sg0)

    def pair(ip, carry):
        for par in range(2):
            i = ip * 2 + par
            idxm, sg, wm, sw = (idx0, sg0, wout0, sw0) if par == 0 \
                else (idx1, sg1, wout1, sw1)
            nidxm, nsg, nwm, nsw = (idx1, sg1, wout1, sw1) if par == 0 \
                else (idx0, sg0, wout0, sw0)

            @pl.when((i + 1 < NBLK) & (i >= 1))
            def _():
                wait_out(nwm, nsw)

            @pl.when(i + 1 < NBLK)
            def _():
                build_idx(i + 1, nidxm)
                start_gathers(nidxm, nwm, nsg)

            wait_gathers(idxm, wm, sg)
            start_out(i, wm, sw)
        return carry

    lax.fori_loop(0, NBLK // 2, pair, 0)
    wait_out(wout0, sw0)
    wait_out(wout1, sw1)


def _gather(xf, scr1d):
    mesh = plsc.VectorSubcoreMesh(core_axis_name="c", subcore_axis_name="s")
    return pl.kernel(
        _gather_body,
        out_type=jax.ShapeDtypeStruct((ROWS2, 128), jnp.float32),
        mesh=mesh,
        scratch_types=[
            pltpu.VMEM((XW,), jnp.int32),
            pltpu.VMEM((D, 128), jnp.int32),
            pltpu.VMEM((D, 128), jnp.int32),
            pltpu.VMEM((D, 128), jnp.float32),
            pltpu.VMEM((D, 128), jnp.float32),
            pltpu.SemaphoreType.DMA,
            pltpu.SemaphoreType.DMA,
            pltpu.SemaphoreType.DMA,
            pltpu.SemaphoreType.DMA,
        ],
        compiler_params=pltpu.CompilerParams(
            use_tc_tiling_on_sc=False, needs_layout_passes=False),
    )(xf, scr1d)


def kernel(X, tables):
    tabT = jnp.transpose(tables, (0, 2, 1))
    tail = tables[:, NTF * 128:, :].reshape(F, VREM * D)
    tailp = jnp.concatenate(
        [tail, jnp.zeros((F, 1024 - VREM * D), jnp.float32)], axis=1
    ).reshape(F * 8, 128)
    scr = _detile(tabT, tailp)
    out2 = _gather(X.reshape(B * F), scr.reshape(-1))
    out = out2.reshape(F, 2, B // 128, 8, 128)
    return out.transpose(2, 4, 0, 1, 3).reshape(B, F, D)
